# baseline (speedup 1.0000x reference)
import functools
from collections import Counter
from math import factorial

import jax
import jax.numpy as jnp
import numpy as np
from jax import lax
from jax.experimental import pallas as pl
from jax.experimental.pallas import tpu as pltpu
from jax.experimental.pallas import tpu_sc as plsc

N_ATOMS = 10000
N_RADIAL = 5
N_BASIS = 7
R_MIN = 0.5
R_MAX = 6.0
N_SPECIES = 119

A_PAD = 10240
BLK = 1024
LANES = 16
NSLOT = 7
NPLANE = NSLOT * LANES

P2 = [(i, j) for i in range(3) for j in range(i, 3)]
P2IDX = {p: n for n, p in enumerate(P2)}
W2C = [1.0 if i == j else 2.0 for (i, j) in P2]
P3 = [(i, j, k) for i in range(3) for j in range(i, 3) for k in range(j, 3)]
P3IDX = {t: n for n, t in enumerate(P3)}


def _w3(t):
    c = Counter(t)
    m = 6
    for v in c.values():
        m //= factorial(v)
    return float(m)


W3C = [_w3(t) for t in P3]

TRI2 = [(i, j) for i in range(N_RADIAL) for j in range(i + 1)]
TRI3 = [(i, j, k) for i in range(N_RADIAL) for j in range(i + 1) for k in range(j + 1)]

N_OUT = 360


def _contr_body(macc_ref, out_ref):
    M = macc_ref[0] + macc_ref[1]

    plane = [M[f] for f in range(NPLANE)]

    def m0(r):
        return plane[r * 16]

    def m1(r, i):
        return plane[r * 16 + 1 + i]

    def m2u(r, p):
        return plane[r * 16 + 4 + p]

    def m3u(r, q):
        if q < 6:
            return plane[r * 16 + 10 + q]
        if r < 4:
            return plane[80 + r * 4 + q - 6]
        return plane[96 + q - 6]

    def m2(r, i, j):
        i, j = sorted((i, j))
        return m2u(r, P2IDX[(i, j)])

    def m3(r, i, j, k):
        i, j, k = sorted((i, j, k))
        return m3u(r, P3IDX[(i, j, k)])

    outs = []
    for r in range(N_RADIAL):
        outs.append(m0(r))
    for (r, s) in TRI2:
        outs.append(sum(m1(r, i) * m1(s, i) for i in range(3)))
    for (r, s) in TRI2:
        outs.append(sum(W2C[p] * m2u(r, p) * m2u(s, p) for p in range(6)))
    for (r, s) in TRI2:
        outs.append(sum(W3C[q] * m3u(r, q) * m3u(s, q) for q in range(10)))
    Bc = {}
    for (r, s, t) in TRI3:
        if (r, s) not in Bc:
            Bc[(r, s)] = [[sum(m2(r, i, j) * m2(s, i, k) for i in range(3))
                           for k in range(3)] for j in range(3)]
        B = Bc[(r, s)]
        outs.append(sum(B[j][k] * m2(t, j, k) for j in range(3) for k in range(3)))
    F5 = [[[sum(m1(r, i) * m2(t, i, j) for i in range(3))
            for j in range(3)] for t in range(N_RADIAL)] for r in range(N_RADIAL)]
    for (r, s) in TRI2:
        for t in range(N_RADIAL):
            outs.append(sum(F5[r][t][j] * m1(s, j) for j in range(3)))
    for (r, s) in TRI2:
        G = [[sum(W2C[p] * m3(r, P2[p][0], P2[p][1], k) * m3(s, P2[p][0], P2[p][1], l)
                  for p in range(6)) for l in range(3)] for k in range(3)]
        for t in range(N_RADIAL):
            outs.append(sum(G[k][l] * m2(t, k, l) for k in range(3) for l in range(3)))
    H = [[[sum(W2C[p] * m3(r, P2[p][0], P2[p][1], k) * m2u(s, p)
               for p in range(6)) for k in range(3)]
          for s in range(N_RADIAL)] for r in range(N_RADIAL)]
    for r in range(N_RADIAL):
        for s in range(N_RADIAL):
            for t in range(N_RADIAL):
                outs.append(sum(H[r][s][k] * m1(t, k) for k in range(3)))

    for f, v in enumerate(outs):
        out_ref[f] = v


def _contract(macc_t):
    return pl.pallas_call(
        _contr_body,
        grid=(A_PAD // BLK,),
        in_specs=[pl.BlockSpec((2, NPLANE, BLK), lambda i: (0, 0, i))],
        out_specs=pl.BlockSpec((N_OUT, BLK), lambda i: (0, i)),
        out_shape=jax.ShapeDtypeStruct((N_OUT, A_PAD), jnp.float32),
    )(macc_t)


NC, NS = 2, 16
NW = NC * NS
E_PAD = 163840
EPW = E_PAD // NW
CHUNK = 128
NGRP = CHUNK // LANES
NCHUNK = EPW // CHUNK
RPS = A_PAD // NS

BETTA = float((N_BASIS ** 2) / (R_MAX ** 2))
RAD_NORM = float((2.0 * BETTA / np.pi) ** 0.25)
EMBED_NORM = float(1.0 / np.sqrt(N_BASIS))
BSTEP = float((R_MAX - R_MIN) / N_BASIS)


def _edge_sc_body(*args):
    (tx, ty, tz, tzf, ie_hbm, je_hbm) = args[:6]
    whbm = list(args[6:9])
    out_hbm = args[9]
    sc = list(args[10:])
    ii_v, jj_v, zp_v = sc[0], sc[1], sc[2]
    ti = sc[3:7]
    tj = sc[7:11]
    geo = sc[11:16]
    w_v = sc[16:19]
    f_v = sc[19:26]
    acc = sc[26:33]
    zb_v = sc[33]
    sem = sc[34]

    c = lax.axis_index("c")
    s = lax.axis_index("s")
    w = s * NC + c
    ebase = w * EPW

    def zloop(i, carry):
        zb_v[i] = jnp.zeros((LANES,), jnp.float32)
        return carry
    lax.fori_loop(0, RPS, zloop, None)
    for k in range(NSLOT):
        pltpu.sync_copy(zb_v, acc[k].at[pl.ds(s * RPS, RPS)])
    plsc.subcore_barrier()

    def i32(x):
        return jnp.int32(x)

    dnums = lax.GatherDimensionNumbers(
        offset_dims=(), collapsed_slice_dims=(0,), start_index_map=(0,))

    def lanegather(v, patt):
        return lax.gather(v, patt[:, None], dnums, (1,),
                          mode=lax.GatherScatterMode.PROMISE_IN_BOUNDS)

    def consts():
        sel = jnp.where
        iota = lax.iota(jnp.int32, LANES)
        zero16 = jnp.zeros((LANES,), jnp.float32)
        ones16 = zero16 + jnp.float32(1.0)
        zi16 = jnp.zeros((LANES,), jnp.int32)
        shifts16 = (jnp.float32(R_MIN)
                    + jnp.float32(BSTEP)
                    * jnp.bitwise_and(iota, 7).astype(jnp.float32))
        rots = [jnp.bitwise_or(jnp.bitwise_and(iota, 8),
                               jnp.bitwise_and(iota + k, 7)) for k in (4, 2, 1)]
        pA = sel(iota < 4, iota,
                 sel(iota < 7, zi16 + 1,
                     sel(iota < 9, zi16 + 2,
                         sel(iota < 10, zi16 + 3, zi16 + 1))))
        pB = sel(iota < 4, zi16,
                 sel(iota < 7, iota - 3,
                     sel(iota < 8, zi16 + 2,
                         sel(iota < 10, zi16 + 3,
                             sel(iota < 13, zi16 + 1,
                                 sel(iota < 15, zi16 + 2, zi16 + 3))))))
        pC = sel(iota < 10, zi16,
                 sel(iota < 13, iota - 9, sel(iota < 14, zi16 + 2, zi16 + 3)))
        pA2 = sel(iota < 3, zi16 + 2, sel(iota < 4, zi16 + 3, zi16 + 4))
        pB2 = sel(iota < 2, zi16 + 2, sel(iota < 4, zi16 + 3, zi16 + 4))
        pC2 = sel(iota < 1, zi16 + 2, sel(iota < 4, zi16 + 3, zi16 + 4))
        shl = [jnp.bitwise_and(iota - 4 * r, 15) for r in (1, 2, 3)]
        return (iota, zero16, ones16, shifts16, rots,
                (pA, pB, pC), (pA2, pB2, pC2), shl)

    def chunk_body(ch, carry):
        base = ebase + ch * CHUNK
        pltpu.sync_copy(ie_hbm.at[pl.ds(base, CHUNK)], ii_v)
        pltpu.sync_copy(je_hbm.at[pl.ds(base, CHUNK)], jj_v)
        cps = [pltpu.async_copy(t, d, sem)
               for t, d in ((tx.at[ii_v], ti[0]), (ty.at[ii_v], ti[1]),
                            (tz.at[ii_v], ti[2]), (tzf.at[ii_v], ti[3]),
                            (tx.at[jj_v], tj[0]), (ty.at[jj_v], tj[1]),
                            (tz.at[jj_v], tj[2]), (tzf.at[jj_v], tj[3]))]
        for cp in cps:
            cp.wait()

        def pass1(g, carry1):
            ds16 = pl.ds(g * LANES, LANES)
            xi, yi, zi, zfi = ti[0][ds16], ti[1][ds16], ti[2][ds16], ti[3][ds16]
            xj, yj, zj, zfj = tj[0][ds16], tj[1][ds16], tj[2][ds16], tj[3][ds16]
            dx, dy, dz = xj - xi, yj - yi, zj - zi
            d2 = jnp.maximum(dx * dx + dy * dy + dz * dz, jnp.float32(1e-24))
            y = lax.bitcast_convert_type(
                i32(0x5F3759DF) - (lax.bitcast_convert_type(d2, jnp.int32) >> 1),
                jnp.float32)
            for _n in range(3):
                y = y * (jnp.float32(1.5) - jnp.float32(0.5) * d2 * y * y)
            dr = d2 * y
            inv = jnp.float32(1.0) / (dr + jnp.float32(1e-5))
            geo[2][ds16] = dx * inv
            geo[3][ds16] = dy * inv
            geo[4][ds16] = dz * inv
            u = jnp.minimum(dr, jnp.float32(R_MAX)) * jnp.float32(np.pi * 0.5 / R_MAX)
            v2 = u * u
            p = jnp.float32(1.0 / 362880.0) + v2 * jnp.float32(-1.0 / 39916800.0)
            p = jnp.float32(-1.0 / 5040.0) + v2 * p
            p = jnp.float32(1.0 / 120.0) + v2 * p
            p = jnp.float32(-1.0 / 6.0) + v2 * p
            sn = u * (jnp.float32(1.0) + v2 * p)
            cc = jnp.float32(1.0) - sn * sn
            iiv = ii_v[ds16]
            jjv = jj_v[ds16]
            scale = jnp.where(iiv != jjv, cc * jnp.float32(RAD_NORM * EMBED_NORM),
                              jnp.zeros((LANES,), jnp.float32))
            geo[0][ds16] = dr
            geo[1][ds16] = scale
            zp = zfj * jnp.float32(N_SPECIES) + zfi
            zp_v[ds16] = zp.astype(jnp.int32)
            return carry1
        lax.fori_loop(0, NGRP, pass1, None)

        wps = [pltpu.async_copy(whbm[r].at[zp_v], w_v[r], sem) for r in range(3)]
        for cp in wps:
            cp.wait()

        def pass2(g, carry2):
            (iota, zero16, ones16, shifts16, rots,
             pABC, pABC2, shl) = consts()
            (pA, pB, pC) = pABC
            (pA2, pB2, pC2) = pABC2
            rot4, rot2, rot1 = rots
            ds16 = pl.ds(g * LANES, LANES)
            drv = geo[0][ds16]
            scv = geo[1][ds16]
            d0v = geo[2][ds16]
            d1v = geo[3][ds16]
            d2v = geo[4][ds16]
            for l in range(LANES):
                e = g * LANES + l
                drb = zero16 + drv[l]
                scb = zero16 + scv[l]
                t = shifts16 - drb
                basisf = jnp.exp(jnp.float32(-BETTA) * t * t) * scb
                d5 = jnp.where(iota == 0, ones16,
                               jnp.where(iota == 1, zero16 + d0v[l],
                                         jnp.where(iota == 2, zero16 + d1v[l],
                                                   jnp.where(iota == 3,
                                                             zero16 + d2v[l],
                                                             zero16))))
                monA = lanegather(d5, pA) * lanegather(d5, pB) * lanegather(d5, pC)
                monB = lanegather(d5, pA2) * lanegather(d5, pB2) * lanegather(d5, pC2)
                zs = []
                for kp in range(3):
                    pr = w_v[kp][e] * basisf
                    pr = pr + lanegather(pr, rot4)
                    pr = pr + lanegather(pr, rot2)
                    pr = pr + lanegather(pr, rot1)
                    zs.append(pr[0])
                    if kp < 2:
                        zs.append(pr[8])
                bvecs = []
                for r in range(N_RADIAL):
                    zb = zero16 + zs[r]
                    f_v[r][e] = zb * monA
                    bvecs.append(zb * monB)
                comb = bvecs[0]
                for r in (1, 2, 3):
                    comb = comb + lanegather(bvecs[r], shl[r - 1])
                f_v[5][e] = comb
                f_v[6][e] = bvecs[4]
            return carry2
        lax.fori_loop(0, NGRP, pass2, None)

        for k in range(NSLOT):
            pltpu.sync_copy(f_v[k], acc[k].at[jj_v], add=True)
        return carry
    lax.fori_loop(0, NCHUNK, chunk_body, None)

    plsc.subcore_barrier()
    for k in range(NSLOT):
        pltpu.sync_copy(acc[k].at[pl.ds(s * RPS, RPS)],
                        out_hbm.at[c, k, pl.ds(s * RPS, RPS)])


@functools.cache
def _edge_sc():
    scr = ([pltpu.VMEM((CHUNK,), jnp.int32)] * 3
           + [pltpu.VMEM((CHUNK,), jnp.float32)] * 8
           + [pltpu.VMEM((CHUNK,), jnp.float32)] * 5
           + [pltpu.VMEM((CHUNK, LANES), jnp.float32)] * 3
           + [pltpu.VMEM((CHUNK, LANES), jnp.float32)] * NSLOT
           + [pltpu.VMEM_SHARED((A_PAD, LANES), jnp.float32)] * NSLOT
           + [pltpu.VMEM((RPS, LANES), jnp.float32),
              pltpu.SemaphoreType.DMA])
    return pl.kernel(
        _edge_sc_body,
        out_type=jax.ShapeDtypeStruct((NC, NSLOT, A_PAD, LANES), jnp.float32),
        mesh=plsc.VectorSubcoreMesh(core_axis_name="c", subcore_axis_name="s",
                                    num_cores=NC, num_subcores=NS),
        compiler_params=pltpu.CompilerParams(use_tc_tiling_on_sc=False),
        scratch_types=scr,
    )


def kernel(R, Z, neighbor_idxs, box, W_emb):
    n_edges = neighbor_idxs.shape[1]
    R = R.astype(jnp.float32)
    tx = jnp.zeros((A_PAD,), jnp.float32).at[:N_ATOMS].set(R[:, 0])
    ty = jnp.zeros((A_PAD,), jnp.float32).at[:N_ATOMS].set(R[:, 1])
    tz = jnp.zeros((A_PAD,), jnp.float32).at[:N_ATOMS].set(R[:, 2])
    tzf = jnp.zeros((A_PAD,), jnp.float32).at[:N_ATOMS].set(Z.astype(jnp.float32))
    ie = jnp.zeros((E_PAD,), jnp.int32).at[:n_edges].set(neighbor_idxs[0])
    je = jnp.zeros((E_PAD,), jnp.int32).at[:n_edges].set(neighbor_idxs[1])
    wflat = W_emb.reshape(N_SPECIES * N_SPECIES, N_RADIAL, N_BASIS).astype(jnp.float32)
    ws = []
    for kp in range(3):
        wp = jnp.zeros((N_SPECIES * N_SPECIES, LANES), jnp.float32)
        wp = wp.at[:, :N_BASIS].set(wflat[:, 2 * kp, :])
        if 2 * kp + 1 < N_RADIAL:
            wp = wp.at[:, 8:8 + N_BASIS].set(wflat[:, 2 * kp + 1, :])
        ws.append(wp)
    macc = _edge_sc()(tx, ty, tz, tzf, ie, je, *ws)
    macc_t = jnp.transpose(macc, (0, 1, 3, 2)).reshape(NC, NPLANE, A_PAD)
    out_t = _contract(macc_t)
    return jnp.transpose(out_t)[:N_ATOMS]


# speedup vs baseline: 1.0619x; 1.0619x over previous
import functools
from collections import Counter
from math import factorial

import jax
import jax.numpy as jnp
import numpy as np
from jax import lax
from jax.experimental import pallas as pl
from jax.experimental.pallas import tpu as pltpu
from jax.experimental.pallas import tpu_sc as plsc

N_ATOMS = 10000
N_RADIAL = 5
N_BASIS = 7
R_MIN = 0.5
R_MAX = 6.0
N_SPECIES = 119

A_PAD = 10240
BLK = 1024
LANES = 16
NSLOT = 7
NPLANE = NSLOT * LANES

P2 = [(i, j) for i in range(3) for j in range(i, 3)]
P2IDX = {p: n for n, p in enumerate(P2)}
W2C = [1.0 if i == j else 2.0 for (i, j) in P2]
P3 = [(i, j, k) for i in range(3) for j in range(i, 3) for k in range(j, 3)]
P3IDX = {t: n for n, t in enumerate(P3)}


def _w3(t):
    c = Counter(t)
    m = 6
    for v in c.values():
        m //= factorial(v)
    return float(m)


W3C = [_w3(t) for t in P3]

TRI2 = [(i, j) for i in range(N_RADIAL) for j in range(i + 1)]
TRI3 = [(i, j, k) for i in range(N_RADIAL) for j in range(i + 1) for k in range(j + 1)]

N_OUT = 360


def _contr_body(macc_ref, out_ref):
    M = macc_ref[0] + macc_ref[1]

    plane = [M[f] for f in range(NPLANE)]

    def m0(r):
        return plane[r * 16]

    def m1(r, i):
        return plane[r * 16 + 1 + i]

    def m2u(r, p):
        return plane[r * 16 + 4 + p]

    def m3u(r, q):
        if q < 6:
            return plane[r * 16 + 10 + q]
        if r < 4:
            return plane[80 + r * 4 + q - 6]
        return plane[96 + q - 6]

    def m2(r, i, j):
        i, j = sorted((i, j))
        return m2u(r, P2IDX[(i, j)])

    def m3(r, i, j, k):
        i, j, k = sorted((i, j, k))
        return m3u(r, P3IDX[(i, j, k)])

    outs = []
    for r in range(N_RADIAL):
        outs.append(m0(r))
    for (r, s) in TRI2:
        outs.append(sum(m1(r, i) * m1(s, i) for i in range(3)))
    for (r, s) in TRI2:
        outs.append(sum(W2C[p] * m2u(r, p) * m2u(s, p) for p in range(6)))
    for (r, s) in TRI2:
        outs.append(sum(W3C[q] * m3u(r, q) * m3u(s, q) for q in range(10)))
    Bc = {}
    for (r, s, t) in TRI3:
        if (r, s) not in Bc:
            Bc[(r, s)] = [[sum(m2(r, i, j) * m2(s, i, k) for i in range(3))
                           for k in range(3)] for j in range(3)]
        B = Bc[(r, s)]
        outs.append(sum(B[j][k] * m2(t, j, k) for j in range(3) for k in range(3)))
    F5 = [[[sum(m1(r, i) * m2(t, i, j) for i in range(3))
            for j in range(3)] for t in range(N_RADIAL)] for r in range(N_RADIAL)]
    for (r, s) in TRI2:
        for t in range(N_RADIAL):
            outs.append(sum(F5[r][t][j] * m1(s, j) for j in range(3)))
    for (r, s) in TRI2:
        G = [[sum(W2C[p] * m3(r, P2[p][0], P2[p][1], k) * m3(s, P2[p][0], P2[p][1], l)
                  for p in range(6)) for l in range(3)] for k in range(3)]
        for t in range(N_RADIAL):
            outs.append(sum(G[k][l] * m2(t, k, l) for k in range(3) for l in range(3)))
    H = [[[sum(W2C[p] * m3(r, P2[p][0], P2[p][1], k) * m2u(s, p)
               for p in range(6)) for k in range(3)]
          for s in range(N_RADIAL)] for r in range(N_RADIAL)]
    for r in range(N_RADIAL):
        for s in range(N_RADIAL):
            for t in range(N_RADIAL):
                outs.append(sum(H[r][s][k] * m1(t, k) for k in range(3)))

    for f, v in enumerate(outs):
        out_ref[f] = v


def _contract(macc_t):
    return pl.pallas_call(
        _contr_body,
        grid=(A_PAD // BLK,),
        in_specs=[pl.BlockSpec((2, NPLANE, BLK), lambda i: (0, 0, i))],
        out_specs=pl.BlockSpec((N_OUT, BLK), lambda i: (0, i)),
        out_shape=jax.ShapeDtypeStruct((N_OUT, A_PAD), jnp.float32),
    )(macc_t)


NC, NS = 2, 16
NW = NC * NS
E_PAD = 163840
EPW = E_PAD // NW
CHUNK = 128
NGRP = CHUNK // LANES
NCHUNK = EPW // CHUNK
RPS = A_PAD // NS

BETTA = float((N_BASIS ** 2) / (R_MAX ** 2))
RAD_NORM = float((2.0 * BETTA / np.pi) ** 0.25)
EMBED_NORM = float(1.0 / np.sqrt(N_BASIS))
BSTEP = float((R_MAX - R_MIN) / N_BASIS)


def _edge_sc_body(*args):
    (tx, ty, tz, tzf, ie_hbm, je_hbm) = args[:6]
    whbm = list(args[6:11])
    out_hbm = args[11]
    sc = list(args[12:])
    ii_v, jj_v, zp_v = sc[0], sc[1], sc[2]
    ti = sc[3:7]
    tj = sc[7:11]
    geo = sc[11:16]
    w_v = sc[16:21]
    f_v = sc[21:28]
    acc = sc[28:35]
    zb_v = sc[35]
    sem = sc[36]

    c = lax.axis_index("c")
    s = lax.axis_index("s")
    w = s * NC + c
    ebase = w * EPW

    def zloop(i, carry):
        zb_v[i] = jnp.zeros((LANES,), jnp.float32)
        return carry
    lax.fori_loop(0, RPS, zloop, None)
    for k in range(NSLOT):
        pltpu.sync_copy(zb_v, acc[k].at[pl.ds(s * RPS, RPS)])
    plsc.subcore_barrier()

    def i32(x):
        return jnp.int32(x)

    dnums = lax.GatherDimensionNumbers(
        offset_dims=(), collapsed_slice_dims=(0,), start_index_map=(0,))

    def lanegather(v, patt):
        return lax.gather(v, patt[:, None], dnums, (1,),
                          mode=lax.GatherScatterMode.PROMISE_IN_BOUNDS)

    def consts():
        sel = jnp.where
        iota = lax.iota(jnp.int32, LANES)
        zero16 = jnp.zeros((LANES,), jnp.float32)
        ones16 = zero16 + jnp.float32(1.0)
        zi16 = jnp.zeros((LANES,), jnp.int32)
        shifts16 = jnp.float32(R_MIN) + jnp.float32(BSTEP) * iota.astype(jnp.float32)
        rots = [jnp.bitwise_and(iota + k, 15) for k in (4, 2, 1)]
        pA = sel(iota < 4, iota,
                 sel(iota < 7, zi16 + 1,
                     sel(iota < 9, zi16 + 2,
                         sel(iota < 10, zi16 + 3, zi16 + 1))))
        pB = sel(iota < 4, zi16,
                 sel(iota < 7, iota - 3,
                     sel(iota < 8, zi16 + 2,
                         sel(iota < 10, zi16 + 3,
                             sel(iota < 13, zi16 + 1,
                                 sel(iota < 15, zi16 + 2, zi16 + 3))))))
        pC = sel(iota < 10, zi16,
                 sel(iota < 13, iota - 9, sel(iota < 14, zi16 + 2, zi16 + 3)))
        pA2 = sel(iota < 3, zi16 + 2, sel(iota < 4, zi16 + 3, zi16 + 4))
        pB2 = sel(iota < 2, zi16 + 2, sel(iota < 4, zi16 + 3, zi16 + 4))
        pC2 = sel(iota < 1, zi16 + 2, sel(iota < 4, zi16 + 3, zi16 + 4))
        shl = [jnp.bitwise_and(iota - 4 * r, 15) for r in (1, 2, 3)]
        return (iota, zero16, ones16, shifts16, rots,
                (pA, pB, pC), (pA2, pB2, pC2), shl)

    def chunk_body(ch, carry):
        base = ebase + ch * CHUNK
        pltpu.sync_copy(ie_hbm.at[pl.ds(base, CHUNK)], ii_v)
        pltpu.sync_copy(je_hbm.at[pl.ds(base, CHUNK)], jj_v)
        cps = [pltpu.async_copy(t, d, sem)
               for t, d in ((tx.at[ii_v], ti[0]), (ty.at[ii_v], ti[1]),
                            (tz.at[ii_v], ti[2]), (tzf.at[ii_v], ti[3]),
                            (tx.at[jj_v], tj[0]), (ty.at[jj_v], tj[1]),
                            (tz.at[jj_v], tj[2]), (tzf.at[jj_v], tj[3]))]
        for cp in cps:
            cp.wait()

        def pass1(g, carry1):
            ds16 = pl.ds(g * LANES, LANES)
            xi, yi, zi, zfi = ti[0][ds16], ti[1][ds16], ti[2][ds16], ti[3][ds16]
            xj, yj, zj, zfj = tj[0][ds16], tj[1][ds16], tj[2][ds16], tj[3][ds16]
            dx, dy, dz = xj - xi, yj - yi, zj - zi
            d2 = jnp.maximum(dx * dx + dy * dy + dz * dz, jnp.float32(1e-24))
            y = lax.bitcast_convert_type(
                i32(0x5F3759DF) - (lax.bitcast_convert_type(d2, jnp.int32) >> 1),
                jnp.float32)
            for _n in range(3):
                y = y * (jnp.float32(1.5) - jnp.float32(0.5) * d2 * y * y)
            dr = d2 * y
            inv = jnp.float32(1.0) / (dr + jnp.float32(1e-5))
            geo[2][ds16] = dx * inv
            geo[3][ds16] = dy * inv
            geo[4][ds16] = dz * inv
            u = jnp.minimum(dr, jnp.float32(R_MAX)) * jnp.float32(np.pi * 0.5 / R_MAX)
            v2 = u * u
            p = jnp.float32(1.0 / 362880.0) + v2 * jnp.float32(-1.0 / 39916800.0)
            p = jnp.float32(-1.0 / 5040.0) + v2 * p
            p = jnp.float32(1.0 / 120.0) + v2 * p
            p = jnp.float32(-1.0 / 6.0) + v2 * p
            sn = u * (jnp.float32(1.0) + v2 * p)
            cc = jnp.float32(1.0) - sn * sn
            iiv = ii_v[ds16]
            jjv = jj_v[ds16]
            scale = jnp.where(iiv != jjv, cc * jnp.float32(RAD_NORM * EMBED_NORM),
                              jnp.zeros((LANES,), jnp.float32))
            geo[0][ds16] = dr
            geo[1][ds16] = scale
            zp = zfj * jnp.float32(N_SPECIES) + zfi
            zp_v[ds16] = zp.astype(jnp.int32)
            return carry1
        lax.fori_loop(0, NGRP, pass1, None)

        wps = [pltpu.async_copy(whbm[r].at[zp_v], w_v[r], sem) for r in range(5)]
        for cp in wps:
            cp.wait()

        def pass2(g, carry2):
            (iota, zero16, ones16, shifts16, rots,
             pABC, pABC2, shl) = consts()
            (pA, pB, pC) = pABC
            (pA2, pB2, pC2) = pABC2
            rot4, rot2, rot1 = rots
            ds16 = pl.ds(g * LANES, LANES)
            drv = geo[0][ds16]
            scv = geo[1][ds16]
            d0v = geo[2][ds16]
            d1v = geo[3][ds16]
            d2v = geo[4][ds16]
            for l in range(LANES):
                e = g * LANES + l
                drb = zero16 + drv[l]
                scb = zero16 + scv[l]
                t = shifts16 - drb
                basisf = jnp.exp(jnp.float32(-BETTA) * t * t) * scb
                d5 = jnp.where(iota == 0, ones16,
                               jnp.where(iota == 1, zero16 + d0v[l],
                                         jnp.where(iota == 2, zero16 + d1v[l],
                                                   jnp.where(iota == 3,
                                                             zero16 + d2v[l],
                                                             zero16))))
                monA = lanegather(d5, pA) * lanegather(d5, pB) * lanegather(d5, pC)
                monB = lanegather(d5, pA2) * lanegather(d5, pB2) * lanegather(d5, pC2)
                bvecs = []
                for r in range(N_RADIAL):
                    pr = w_v[r][e] * basisf
                    pr = pr + lanegather(pr, rot4)
                    pr = pr + lanegather(pr, rot2)
                    pr = pr + lanegather(pr, rot1)
                    zb = zero16 + pr[0]
                    f_v[r][e] = zb * monA
                    bvecs.append(zb * monB)
                comb = bvecs[0]
                for r in (1, 2, 3):
                    comb = comb + lanegather(bvecs[r], shl[r - 1])
                f_v[5][e] = comb
                f_v[6][e] = bvecs[4]
            return carry2
        lax.fori_loop(0, NGRP, pass2, None)

        sps = [pltpu.async_copy(f_v[k], acc[k].at[jj_v], sem, add=True)
               for k in range(NSLOT)]
        for cp in sps:
            cp.wait()
        return carry
    lax.fori_loop(0, NCHUNK, chunk_body, None)

    plsc.subcore_barrier()
    for k in range(NSLOT):
        pltpu.sync_copy(acc[k].at[pl.ds(s * RPS, RPS)],
                        out_hbm.at[c, k, pl.ds(s * RPS, RPS)])


@functools.cache
def _edge_sc():
    scr = ([pltpu.VMEM((CHUNK,), jnp.int32)] * 3
           + [pltpu.VMEM((CHUNK,), jnp.float32)] * 8
           + [pltpu.VMEM((CHUNK,), jnp.float32)] * 5
           + [pltpu.VMEM((CHUNK, LANES), jnp.float32)] * 5
           + [pltpu.VMEM((CHUNK, LANES), jnp.float32)] * NSLOT
           + [pltpu.VMEM_SHARED((A_PAD, LANES), jnp.float32)] * NSLOT
           + [pltpu.VMEM((RPS, LANES), jnp.float32),
              pltpu.SemaphoreType.DMA])
    return pl.kernel(
        _edge_sc_body,
        out_type=jax.ShapeDtypeStruct((NC, NSLOT, A_PAD, LANES), jnp.float32),
        mesh=plsc.VectorSubcoreMesh(core_axis_name="c", subcore_axis_name="s",
                                    num_cores=NC, num_subcores=NS),
        compiler_params=pltpu.CompilerParams(use_tc_tiling_on_sc=False),
        scratch_types=scr,
    )


def kernel(R, Z, neighbor_idxs, box, W_emb):
    n_edges = neighbor_idxs.shape[1]
    R = R.astype(jnp.float32)
    tx = jnp.zeros((A_PAD,), jnp.float32).at[:N_ATOMS].set(R[:, 0])
    ty = jnp.zeros((A_PAD,), jnp.float32).at[:N_ATOMS].set(R[:, 1])
    tz = jnp.zeros((A_PAD,), jnp.float32).at[:N_ATOMS].set(R[:, 2])
    tzf = jnp.zeros((A_PAD,), jnp.float32).at[:N_ATOMS].set(Z.astype(jnp.float32))
    ie = jnp.zeros((E_PAD,), jnp.int32).at[:n_edges].set(neighbor_idxs[0])
    je = jnp.zeros((E_PAD,), jnp.int32).at[:n_edges].set(neighbor_idxs[1])
    wflat = W_emb.reshape(N_SPECIES * N_SPECIES, N_RADIAL, N_BASIS).astype(jnp.float32)
    ws = [jnp.zeros((N_SPECIES * N_SPECIES, LANES), jnp.float32)
          .at[:, :N_BASIS].set(wflat[:, r, :]) for r in range(N_RADIAL)]
    macc = _edge_sc()(tx, ty, tz, tzf, ie, je, *ws)
    macc_t = jnp.transpose(macc, (0, 1, 3, 2)).reshape(NC, NPLANE, A_PAD)
    out_t = _contract(macc_t)
    return jnp.transpose(out_t)[:N_ATOMS]


# speedup vs baseline: 1.0670x; 1.0048x over previous
import functools
from collections import Counter
from math import factorial

import jax
import jax.numpy as jnp
import numpy as np
from jax import lax
from jax.experimental import pallas as pl
from jax.experimental.pallas import tpu as pltpu
from jax.experimental.pallas import tpu_sc as plsc

N_ATOMS = 10000
N_RADIAL = 5
N_BASIS = 7
R_MIN = 0.5
R_MAX = 6.0
N_SPECIES = 119

A_PAD = 10240
BLK = 1024
LANES = 16
NSLOT = 7
NPLANE = NSLOT * LANES

P2 = [(i, j) for i in range(3) for j in range(i, 3)]
P2IDX = {p: n for n, p in enumerate(P2)}
W2C = [1.0 if i == j else 2.0 for (i, j) in P2]
P3 = [(i, j, k) for i in range(3) for j in range(i, 3) for k in range(j, 3)]
P3IDX = {t: n for n, t in enumerate(P3)}


def _w3(t):
    c = Counter(t)
    m = 6
    for v in c.values():
        m //= factorial(v)
    return float(m)


W3C = [_w3(t) for t in P3]

TRI2 = [(i, j) for i in range(N_RADIAL) for j in range(i + 1)]
TRI3 = [(i, j, k) for i in range(N_RADIAL) for j in range(i + 1) for k in range(j + 1)]

N_OUT = 360


def _contr_body(macc_ref, out_ref):
    M = macc_ref[0] + macc_ref[1]

    plane = [M[f] for f in range(NPLANE)]

    def m0(r):
        return plane[r * 16]

    def m1(r, i):
        return plane[r * 16 + 1 + i]

    def m2u(r, p):
        return plane[r * 16 + 4 + p]

    def m3u(r, q):
        if q < 6:
            return plane[r * 16 + 10 + q]
        if r < 4:
            return plane[80 + r * 4 + q - 6]
        return plane[96 + q - 6]

    def m2(r, i, j):
        i, j = sorted((i, j))
        return m2u(r, P2IDX[(i, j)])

    def m3(r, i, j, k):
        i, j, k = sorted((i, j, k))
        return m3u(r, P3IDX[(i, j, k)])

    outs = []
    for r in range(N_RADIAL):
        outs.append(m0(r))
    for (r, s) in TRI2:
        outs.append(sum(m1(r, i) * m1(s, i) for i in range(3)))
    for (r, s) in TRI2:
        outs.append(sum(W2C[p] * m2u(r, p) * m2u(s, p) for p in range(6)))
    for (r, s) in TRI2:
        outs.append(sum(W3C[q] * m3u(r, q) * m3u(s, q) for q in range(10)))
    Bc = {}
    for (r, s, t) in TRI3:
        if (r, s) not in Bc:
            Bc[(r, s)] = [[sum(m2(r, i, j) * m2(s, i, k) for i in range(3))
                           for k in range(3)] for j in range(3)]
        B = Bc[(r, s)]
        outs.append(sum(B[j][k] * m2(t, j, k) for j in range(3) for k in range(3)))
    F5 = [[[sum(m1(r, i) * m2(t, i, j) for i in range(3))
            for j in range(3)] for t in range(N_RADIAL)] for r in range(N_RADIAL)]
    for (r, s) in TRI2:
        for t in range(N_RADIAL):
            outs.append(sum(F5[r][t][j] * m1(s, j) for j in range(3)))
    for (r, s) in TRI2:
        G = [[sum(W2C[p] * m3(r, P2[p][0], P2[p][1], k) * m3(s, P2[p][0], P2[p][1], l)
                  for p in range(6)) for l in range(3)] for k in range(3)]
        for t in range(N_RADIAL):
            outs.append(sum(G[k][l] * m2(t, k, l) for k in range(3) for l in range(3)))
    H = [[[sum(W2C[p] * m3(r, P2[p][0], P2[p][1], k) * m2u(s, p)
               for p in range(6)) for k in range(3)]
          for s in range(N_RADIAL)] for r in range(N_RADIAL)]
    for r in range(N_RADIAL):
        for s in range(N_RADIAL):
            for t in range(N_RADIAL):
                outs.append(sum(H[r][s][k] * m1(t, k) for k in range(3)))

    for f, v in enumerate(outs):
        out_ref[f] = v


def _contract(macc_t):
    return pl.pallas_call(
        _contr_body,
        grid=(A_PAD // BLK,),
        in_specs=[pl.BlockSpec((2, NPLANE, BLK), lambda i: (0, 0, i))],
        out_specs=pl.BlockSpec((N_OUT, BLK), lambda i: (0, i)),
        out_shape=jax.ShapeDtypeStruct((N_OUT, A_PAD), jnp.float32),
    )(macc_t)


NC, NS = 2, 16
NW = NC * NS
E_PAD = 163840
EPW = E_PAD // NW
CHUNK = 128
NGRP = CHUNK // LANES
NCHUNK = EPW // CHUNK
RPS = A_PAD // NS

BETTA = float((N_BASIS ** 2) / (R_MAX ** 2))
RAD_NORM = float((2.0 * BETTA / np.pi) ** 0.25)
EMBED_NORM = float(1.0 / np.sqrt(N_BASIS))
BSTEP = float((R_MAX - R_MIN) / N_BASIS)


def _edge_sc_body(*args):
    (tx, ty, tz, tzf, ie_hbm, je_hbm) = args[:6]
    whbm = list(args[6:11])
    out_hbm = args[11]
    sc = list(args[12:])
    ii_v, jj_v, zp_v = sc[0], sc[1], sc[2]
    ti = sc[3:7]
    tj = sc[7:11]
    geo = sc[11:16]
    w_v = sc[16:21]
    f_v = sc[21:28]
    acc = sc[28:35]
    zb_v = sc[35]
    sem = sc[36]

    c = lax.axis_index("c")
    s = lax.axis_index("s")
    w = s * NC + c
    ebase = w * EPW

    def zloop(i, carry):
        zb_v[i] = jnp.zeros((LANES,), jnp.float32)
        return carry
    lax.fori_loop(0, RPS, zloop, None)
    for k in range(NSLOT):
        pltpu.sync_copy(zb_v, acc[k].at[pl.ds(s * RPS, RPS)])
    plsc.subcore_barrier()

    def i32(x):
        return jnp.int32(x)

    dnums = lax.GatherDimensionNumbers(
        offset_dims=(), collapsed_slice_dims=(0,), start_index_map=(0,))

    def lanegather(v, patt):
        return lax.gather(v, patt[:, None], dnums, (1,),
                          mode=lax.GatherScatterMode.PROMISE_IN_BOUNDS)

    def consts():
        sel = jnp.where
        iota = lax.iota(jnp.int32, LANES)
        zero16 = jnp.zeros((LANES,), jnp.float32)
        ones16 = zero16 + jnp.float32(1.0)
        zi16 = jnp.zeros((LANES,), jnp.int32)
        shifts16 = jnp.float32(R_MIN) + jnp.float32(BSTEP) * iota.astype(jnp.float32)
        rots = [jnp.bitwise_and(iota + k, 15) for k in (4, 2, 1)]
        pA = sel(iota < 4, iota,
                 sel(iota < 7, zi16 + 1,
                     sel(iota < 9, zi16 + 2,
                         sel(iota < 10, zi16 + 3, zi16 + 1))))
        pB = sel(iota < 4, zi16,
                 sel(iota < 7, iota - 3,
                     sel(iota < 8, zi16 + 2,
                         sel(iota < 10, zi16 + 3,
                             sel(iota < 13, zi16 + 1,
                                 sel(iota < 15, zi16 + 2, zi16 + 3))))))
        pC = sel(iota < 10, zi16,
                 sel(iota < 13, iota - 9, sel(iota < 14, zi16 + 2, zi16 + 3)))
        pA2 = sel(iota < 3, zi16 + 2, sel(iota < 4, zi16 + 3, zi16 + 4))
        pB2 = sel(iota < 2, zi16 + 2, sel(iota < 4, zi16 + 3, zi16 + 4))
        pC2 = sel(iota < 1, zi16 + 2, sel(iota < 4, zi16 + 3, zi16 + 4))
        shl = [jnp.bitwise_and(iota - 4 * r, 15) for r in (1, 2, 3)]
        return (iota, zero16, ones16, shifts16, rots,
                (pA, pB, pC), (pA2, pB2, pC2), shl)

    def chunk_body(ch, carry):
        base = ebase + ch * CHUNK
        pltpu.sync_copy(ie_hbm.at[pl.ds(base, CHUNK)], ii_v)
        pltpu.sync_copy(je_hbm.at[pl.ds(base, CHUNK)], jj_v)
        cps = [pltpu.async_copy(t, d, sem)
               for t, d in ((tx.at[ii_v], ti[0]), (ty.at[ii_v], ti[1]),
                            (tz.at[ii_v], ti[2]), (tzf.at[ii_v], ti[3]),
                            (tx.at[jj_v], tj[0]), (ty.at[jj_v], tj[1]),
                            (tz.at[jj_v], tj[2]), (tzf.at[jj_v], tj[3]))]
        for cp in cps:
            cp.wait()

        def pass1(g, carry1):
            ds16 = pl.ds(g * LANES, LANES)
            xi, yi, zi, zfi = ti[0][ds16], ti[1][ds16], ti[2][ds16], ti[3][ds16]
            xj, yj, zj, zfj = tj[0][ds16], tj[1][ds16], tj[2][ds16], tj[3][ds16]
            dx, dy, dz = xj - xi, yj - yi, zj - zi
            d2 = jnp.maximum(dx * dx + dy * dy + dz * dz, jnp.float32(1e-24))
            y = lax.bitcast_convert_type(
                i32(0x5F3759DF) - (lax.bitcast_convert_type(d2, jnp.int32) >> 1),
                jnp.float32)
            for _n in range(3):
                y = y * (jnp.float32(1.5) - jnp.float32(0.5) * d2 * y * y)
            dr = d2 * y
            inv = jnp.float32(1.0) / (dr + jnp.float32(1e-5))
            geo[2][ds16] = dx * inv
            geo[3][ds16] = dy * inv
            geo[4][ds16] = dz * inv
            u = jnp.minimum(dr, jnp.float32(R_MAX)) * jnp.float32(np.pi * 0.5 / R_MAX)
            v2 = u * u
            p = jnp.float32(1.0 / 362880.0) + v2 * jnp.float32(-1.0 / 39916800.0)
            p = jnp.float32(-1.0 / 5040.0) + v2 * p
            p = jnp.float32(1.0 / 120.0) + v2 * p
            p = jnp.float32(-1.0 / 6.0) + v2 * p
            sn = u * (jnp.float32(1.0) + v2 * p)
            cc = jnp.float32(1.0) - sn * sn
            iiv = ii_v[ds16]
            jjv = jj_v[ds16]
            scale = jnp.where(iiv != jjv, cc * jnp.float32(RAD_NORM * EMBED_NORM),
                              jnp.zeros((LANES,), jnp.float32))
            geo[0][ds16] = dr
            geo[1][ds16] = scale
            zp = zfj * jnp.float32(N_SPECIES) + zfi
            zp_v[ds16] = zp.astype(jnp.int32)
            return carry1
        lax.fori_loop(0, NGRP, pass1, None)

        wps = [pltpu.async_copy(whbm[r].at[zp_v], w_v[r], sem) for r in range(5)]
        for cp in wps:
            cp.wait()

        def pass2(g, carry2):
            (iota, zero16, ones16, shifts16, rots,
             pABC, pABC2, shl) = consts()
            (pA, pB, pC) = pABC
            (pA2, pB2, pC2) = pABC2
            rot4, rot2, rot1 = rots
            ds16 = pl.ds(g * LANES, LANES)
            drv = geo[0][ds16]
            scv = geo[1][ds16]
            d0v = geo[2][ds16]
            d1v = geo[3][ds16]
            d2v = geo[4][ds16]
            zi = jnp.bitwise_and(iota, 0)
            for l in range(LANES):
                e = g * LANES + l
                patl = zi + l
                drb = lanegather(drv, patl)
                scb = lanegather(scv, patl)
                t = shifts16 - drb
                basisf = jnp.exp(jnp.float32(-BETTA) * t * t) * scb
                d5 = jnp.where(iota == 0, ones16,
                               jnp.where(iota == 1, lanegather(d0v, patl),
                                         jnp.where(iota == 2, lanegather(d1v, patl),
                                                   jnp.where(iota == 3,
                                                             lanegather(d2v, patl),
                                                             zero16))))
                monA = lanegather(d5, pA) * lanegather(d5, pB) * lanegather(d5, pC)
                monB = lanegather(d5, pA2) * lanegather(d5, pB2) * lanegather(d5, pC2)
                bvecs = []
                for r in range(N_RADIAL):
                    pr = w_v[r][e] * basisf
                    pr = pr + lanegather(pr, rot4)
                    pr = pr + lanegather(pr, rot2)
                    pr = pr + lanegather(pr, rot1)
                    zb = lanegather(pr, zi)
                    f_v[r][e] = zb * monA
                    bvecs.append(zb * monB)
                comb = bvecs[0]
                for r in (1, 2, 3):
                    comb = comb + lanegather(bvecs[r], shl[r - 1])
                f_v[5][e] = comb
                f_v[6][e] = bvecs[4]
            return carry2
        lax.fori_loop(0, NGRP, pass2, None)

        sps = [pltpu.async_copy(f_v[k], acc[k].at[jj_v], sem, add=True)
               for k in range(NSLOT)]
        for cp in sps:
            cp.wait()
        return carry
    lax.fori_loop(0, NCHUNK, chunk_body, None)

    plsc.subcore_barrier()
    for k in range(NSLOT):
        pltpu.sync_copy(acc[k].at[pl.ds(s * RPS, RPS)],
                        out_hbm.at[c, k, pl.ds(s * RPS, RPS)])


@functools.cache
def _edge_sc():
    scr = ([pltpu.VMEM((CHUNK,), jnp.int32)] * 3
           + [pltpu.VMEM((CHUNK,), jnp.float32)] * 8
           + [pltpu.VMEM((CHUNK,), jnp.float32)] * 5
           + [pltpu.VMEM((CHUNK, LANES), jnp.float32)] * 5
           + [pltpu.VMEM((CHUNK, LANES), jnp.float32)] * NSLOT
           + [pltpu.VMEM_SHARED((A_PAD, LANES), jnp.float32)] * NSLOT
           + [pltpu.VMEM((RPS, LANES), jnp.float32),
              pltpu.SemaphoreType.DMA])
    return pl.kernel(
        _edge_sc_body,
        out_type=jax.ShapeDtypeStruct((NC, NSLOT, A_PAD, LANES), jnp.float32),
        mesh=plsc.VectorSubcoreMesh(core_axis_name="c", subcore_axis_name="s",
                                    num_cores=NC, num_subcores=NS),
        compiler_params=pltpu.CompilerParams(use_tc_tiling_on_sc=False),
        scratch_types=scr,
    )


def kernel(R, Z, neighbor_idxs, box, W_emb):
    n_edges = neighbor_idxs.shape[1]
    R = R.astype(jnp.float32)
    tx = jnp.zeros((A_PAD,), jnp.float32).at[:N_ATOMS].set(R[:, 0])
    ty = jnp.zeros((A_PAD,), jnp.float32).at[:N_ATOMS].set(R[:, 1])
    tz = jnp.zeros((A_PAD,), jnp.float32).at[:N_ATOMS].set(R[:, 2])
    tzf = jnp.zeros((A_PAD,), jnp.float32).at[:N_ATOMS].set(Z.astype(jnp.float32))
    ie = jnp.zeros((E_PAD,), jnp.int32).at[:n_edges].set(neighbor_idxs[0])
    je = jnp.zeros((E_PAD,), jnp.int32).at[:n_edges].set(neighbor_idxs[1])
    wflat = W_emb.reshape(N_SPECIES * N_SPECIES, N_RADIAL, N_BASIS).astype(jnp.float32)
    ws = [jnp.zeros((N_SPECIES * N_SPECIES, LANES), jnp.float32)
          .at[:, :N_BASIS].set(wflat[:, r, :]) for r in range(N_RADIAL)]
    macc = _edge_sc()(tx, ty, tz, tzf, ie, je, *ws)
    macc_t = jnp.transpose(macc, (0, 1, 3, 2)).reshape(NC, NPLANE, A_PAD)
    out_t = _contract(macc_t)
    return jnp.transpose(out_t)[:N_ATOMS]


# speedup vs baseline: 1.0912x; 1.0227x over previous
import functools
from collections import Counter
from math import factorial

import jax
import jax.numpy as jnp
import numpy as np
from jax import lax
from jax.experimental import pallas as pl
from jax.experimental.pallas import tpu as pltpu
from jax.experimental.pallas import tpu_sc as plsc

N_ATOMS = 10000
N_RADIAL = 5
N_BASIS = 7
R_MIN = 0.5
R_MAX = 6.0
N_SPECIES = 119

A_PAD = 10240
BLK = 1024
LANES = 16
NSLOT = 7
NPLANE = NSLOT * LANES

P2 = [(i, j) for i in range(3) for j in range(i, 3)]
P2IDX = {p: n for n, p in enumerate(P2)}
W2C = [1.0 if i == j else 2.0 for (i, j) in P2]
P3 = [(i, j, k) for i in range(3) for j in range(i, 3) for k in range(j, 3)]
P3IDX = {t: n for n, t in enumerate(P3)}


def _w3(t):
    c = Counter(t)
    m = 6
    for v in c.values():
        m //= factorial(v)
    return float(m)


W3C = [_w3(t) for t in P3]

TRI2 = [(i, j) for i in range(N_RADIAL) for j in range(i + 1)]
TRI3 = [(i, j, k) for i in range(N_RADIAL) for j in range(i + 1) for k in range(j + 1)]

N_OUT = 360


def _contr_body(macc_ref, out_ref):
    M = macc_ref[0] + macc_ref[1]
    plane = []
    for k in range(NSLOT):
        Mt = jnp.transpose(M[k])
        for l in range(LANES):
            plane.append(Mt[l])

    def m0(r):
        return plane[r * 16]

    def m1(r, i):
        return plane[r * 16 + 1 + i]

    def m2u(r, p):
        return plane[r * 16 + 4 + p]

    def m3u(r, q):
        if q < 6:
            return plane[r * 16 + 10 + q]
        if r < 4:
            return plane[80 + r * 4 + q - 6]
        return plane[96 + q - 6]

    def m2(r, i, j):
        i, j = sorted((i, j))
        return m2u(r, P2IDX[(i, j)])

    def m3(r, i, j, k):
        i, j, k = sorted((i, j, k))
        return m3u(r, P3IDX[(i, j, k)])

    outs = []
    for r in range(N_RADIAL):
        outs.append(m0(r))
    for (r, s) in TRI2:
        outs.append(sum(m1(r, i) * m1(s, i) for i in range(3)))
    for (r, s) in TRI2:
        outs.append(sum(W2C[p] * m2u(r, p) * m2u(s, p) for p in range(6)))
    for (r, s) in TRI2:
        outs.append(sum(W3C[q] * m3u(r, q) * m3u(s, q) for q in range(10)))
    Bc = {}
    for (r, s, t) in TRI3:
        if (r, s) not in Bc:
            Bc[(r, s)] = [[sum(m2(r, i, j) * m2(s, i, k) for i in range(3))
                           for k in range(3)] for j in range(3)]
        B = Bc[(r, s)]
        outs.append(sum(B[j][k] * m2(t, j, k) for j in range(3) for k in range(3)))
    F5 = [[[sum(m1(r, i) * m2(t, i, j) for i in range(3))
            for j in range(3)] for t in range(N_RADIAL)] for r in range(N_RADIAL)]
    for (r, s) in TRI2:
        for t in range(N_RADIAL):
            outs.append(sum(F5[r][t][j] * m1(s, j) for j in range(3)))
    for (r, s) in TRI2:
        G = [[sum(W2C[p] * m3(r, P2[p][0], P2[p][1], k) * m3(s, P2[p][0], P2[p][1], l)
                  for p in range(6)) for l in range(3)] for k in range(3)]
        for t in range(N_RADIAL):
            outs.append(sum(G[k][l] * m2(t, k, l) for k in range(3) for l in range(3)))
    H = [[[sum(W2C[p] * m3(r, P2[p][0], P2[p][1], k) * m2u(s, p)
               for p in range(6)) for k in range(3)]
          for s in range(N_RADIAL)] for r in range(N_RADIAL)]
    for r in range(N_RADIAL):
        for s in range(N_RADIAL):
            for t in range(N_RADIAL):
                outs.append(sum(H[r][s][k] * m1(t, k) for k in range(3)))

    for f, v in enumerate(outs):
        out_ref[f] = v


def _contract(macc):
    return pl.pallas_call(
        _contr_body,
        grid=(A_PAD // BLK,),
        in_specs=[pl.BlockSpec((2, NSLOT, BLK, 16), lambda i: (0, 0, i, 0))],
        out_specs=pl.BlockSpec((N_OUT, BLK), lambda i: (0, i)),
        out_shape=jax.ShapeDtypeStruct((N_OUT, A_PAD), jnp.float32),
    )(macc)


NC, NS = 2, 16
NW = NC * NS
E_PAD = 163840
EPW = E_PAD // NW
CHUNK = 128
NGRP = CHUNK // LANES
NCHUNK = EPW // CHUNK
RPS = A_PAD // NS

BETTA = float((N_BASIS ** 2) / (R_MAX ** 2))
RAD_NORM = float((2.0 * BETTA / np.pi) ** 0.25)
EMBED_NORM = float(1.0 / np.sqrt(N_BASIS))
BSTEP = float((R_MAX - R_MIN) / N_BASIS)


def _edge_sc_body(*args):
    (tx, ty, tz, tzf, ie_hbm, je_hbm) = args[:6]
    whbm = list(args[6:11])
    out_hbm = args[11]
    sc = list(args[12:])
    ii_v, jj_v, zp_v = sc[0], sc[1], sc[2]
    ti = sc[3:7]
    tj = sc[7:11]
    geo = sc[11:16]
    w_v = sc[16:21]
    f_v = sc[21:28]
    acc = sc[28:35]
    zb_v = sc[35]
    sem = sc[36]

    c = lax.axis_index("c")
    s = lax.axis_index("s")
    w = s * NC + c
    ebase = w * EPW

    def zloop(i, carry):
        zb_v[i] = jnp.zeros((LANES,), jnp.float32)
        return carry
    lax.fori_loop(0, RPS, zloop, None)
    for k in range(NSLOT):
        pltpu.sync_copy(zb_v, acc[k].at[pl.ds(s * RPS, RPS)])
    plsc.subcore_barrier()

    def i32(x):
        return jnp.int32(x)

    dnums = lax.GatherDimensionNumbers(
        offset_dims=(), collapsed_slice_dims=(0,), start_index_map=(0,))

    def lanegather(v, patt):
        return lax.gather(v, patt[:, None], dnums, (1,),
                          mode=lax.GatherScatterMode.PROMISE_IN_BOUNDS)

    def consts():
        sel = jnp.where
        iota = lax.iota(jnp.int32, LANES)
        zero16 = jnp.zeros((LANES,), jnp.float32)
        ones16 = zero16 + jnp.float32(1.0)
        zi16 = jnp.zeros((LANES,), jnp.int32)
        shifts16 = jnp.float32(R_MIN) + jnp.float32(BSTEP) * iota.astype(jnp.float32)
        rots = [jnp.bitwise_and(iota + k, 15) for k in (4, 2, 1)]
        pA = sel(iota < 4, iota,
                 sel(iota < 7, zi16 + 1,
                     sel(iota < 9, zi16 + 2,
                         sel(iota < 10, zi16 + 3, zi16 + 1))))
        pB = sel(iota < 4, zi16,
                 sel(iota < 7, iota - 3,
                     sel(iota < 8, zi16 + 2,
                         sel(iota < 10, zi16 + 3,
                             sel(iota < 13, zi16 + 1,
                                 sel(iota < 15, zi16 + 2, zi16 + 3))))))
        pC = sel(iota < 10, zi16,
                 sel(iota < 13, iota - 9, sel(iota < 14, zi16 + 2, zi16 + 3)))
        pA2 = sel(iota < 3, zi16 + 2, sel(iota < 4, zi16 + 3, zi16 + 4))
        pB2 = sel(iota < 2, zi16 + 2, sel(iota < 4, zi16 + 3, zi16 + 4))
        pC2 = sel(iota < 1, zi16 + 2, sel(iota < 4, zi16 + 3, zi16 + 4))
        shl = [jnp.bitwise_and(iota - 4 * r, 15) for r in (1, 2, 3)]
        return (iota, zero16, ones16, shifts16, rots,
                (pA, pB, pC), (pA2, pB2, pC2), shl)

    def chunk_body(ch, carry):
        base = ebase + ch * CHUNK
        pltpu.sync_copy(ie_hbm.at[pl.ds(base, CHUNK)], ii_v)
        pltpu.sync_copy(je_hbm.at[pl.ds(base, CHUNK)], jj_v)
        cps = [pltpu.async_copy(t, d, sem)
               for t, d in ((tx.at[ii_v], ti[0]), (ty.at[ii_v], ti[1]),
                            (tz.at[ii_v], ti[2]), (tzf.at[ii_v], ti[3]),
                            (tx.at[jj_v], tj[0]), (ty.at[jj_v], tj[1]),
                            (tz.at[jj_v], tj[2]), (tzf.at[jj_v], tj[3]))]
        for cp in cps:
            cp.wait()

        def pass1(g, carry1):
            ds16 = pl.ds(g * LANES, LANES)
            xi, yi, zi, zfi = ti[0][ds16], ti[1][ds16], ti[2][ds16], ti[3][ds16]
            xj, yj, zj, zfj = tj[0][ds16], tj[1][ds16], tj[2][ds16], tj[3][ds16]
            dx, dy, dz = xj - xi, yj - yi, zj - zi
            d2 = jnp.maximum(dx * dx + dy * dy + dz * dz, jnp.float32(1e-24))
            y = lax.bitcast_convert_type(
                i32(0x5F3759DF) - (lax.bitcast_convert_type(d2, jnp.int32) >> 1),
                jnp.float32)
            for _n in range(3):
                y = y * (jnp.float32(1.5) - jnp.float32(0.5) * d2 * y * y)
            dr = d2 * y
            inv = jnp.float32(1.0) / (dr + jnp.float32(1e-5))
            geo[2][ds16] = dx * inv
            geo[3][ds16] = dy * inv
            geo[4][ds16] = dz * inv
            u = jnp.minimum(dr, jnp.float32(R_MAX)) * jnp.float32(np.pi * 0.5 / R_MAX)
            v2 = u * u
            p = jnp.float32(1.0 / 362880.0) + v2 * jnp.float32(-1.0 / 39916800.0)
            p = jnp.float32(-1.0 / 5040.0) + v2 * p
            p = jnp.float32(1.0 / 120.0) + v2 * p
            p = jnp.float32(-1.0 / 6.0) + v2 * p
            sn = u * (jnp.float32(1.0) + v2 * p)
            cc = jnp.float32(1.0) - sn * sn
            iiv = ii_v[ds16]
            jjv = jj_v[ds16]
            scale = jnp.where(iiv != jjv, cc * jnp.float32(RAD_NORM * EMBED_NORM),
                              jnp.zeros((LANES,), jnp.float32))
            geo[0][ds16] = dr
            geo[1][ds16] = scale
            zp = zfj * jnp.float32(N_SPECIES) + zfi
            zp_v[ds16] = zp.astype(jnp.int32)
            return carry1
        lax.fori_loop(0, NGRP, pass1, None)

        wps = [pltpu.async_copy(whbm[r].at[zp_v], w_v[r], sem) for r in range(5)]
        for cp in wps:
            cp.wait()

        def pass2(g, carry2):
            (iota, zero16, ones16, shifts16, rots,
             pABC, pABC2, shl) = consts()
            (pA, pB, pC) = pABC
            (pA2, pB2, pC2) = pABC2
            rot4, rot2, rot1 = rots
            ds16 = pl.ds(g * LANES, LANES)
            drv = geo[0][ds16]
            scv = geo[1][ds16]
            d0v = geo[2][ds16]
            d1v = geo[3][ds16]
            d2v = geo[4][ds16]
            zi = jnp.bitwise_and(iota, 0)
            for l in range(LANES):
                e = g * LANES + l
                patl = zi + l
                drb = lanegather(drv, patl)
                scb = lanegather(scv, patl)
                t = shifts16 - drb
                basisf = jnp.exp(jnp.float32(-BETTA) * t * t) * scb
                d5 = jnp.where(iota == 0, ones16,
                               jnp.where(iota == 1, lanegather(d0v, patl),
                                         jnp.where(iota == 2, lanegather(d1v, patl),
                                                   jnp.where(iota == 3,
                                                             lanegather(d2v, patl),
                                                             zero16))))
                monA = lanegather(d5, pA) * lanegather(d5, pB) * lanegather(d5, pC)
                monB = lanegather(d5, pA2) * lanegather(d5, pB2) * lanegather(d5, pC2)
                bvecs = []
                for r in range(N_RADIAL):
                    pr = w_v[r][e] * basisf
                    pr = pr + lanegather(pr, rot4)
                    pr = pr + lanegather(pr, rot2)
                    pr = pr + lanegather(pr, rot1)
                    zb = lanegather(pr, zi)
                    f_v[r][e] = zb * monA
                    bvecs.append(zb * monB)
                comb = bvecs[0]
                for r in (1, 2, 3):
                    comb = comb + lanegather(bvecs[r], shl[r - 1])
                f_v[5][e] = comb
                f_v[6][e] = bvecs[4]
            return carry2
        lax.fori_loop(0, NGRP, pass2, None)

        sps = [pltpu.async_copy(f_v[k], acc[k].at[jj_v], sem, add=True)
               for k in range(NSLOT)]
        for cp in sps:
            cp.wait()
        return carry
    lax.fori_loop(0, NCHUNK, chunk_body, None)

    plsc.subcore_barrier()
    for k in range(NSLOT):
        pltpu.sync_copy(acc[k].at[pl.ds(s * RPS, RPS)],
                        out_hbm.at[c, k, pl.ds(s * RPS, RPS)])


@functools.cache
def _edge_sc():
    scr = ([pltpu.VMEM((CHUNK,), jnp.int32)] * 3
           + [pltpu.VMEM((CHUNK,), jnp.float32)] * 8
           + [pltpu.VMEM((CHUNK,), jnp.float32)] * 5
           + [pltpu.VMEM((CHUNK, LANES), jnp.float32)] * 5
           + [pltpu.VMEM((CHUNK, LANES), jnp.float32)] * NSLOT
           + [pltpu.VMEM_SHARED((A_PAD, LANES), jnp.float32)] * NSLOT
           + [pltpu.VMEM((RPS, LANES), jnp.float32),
              pltpu.SemaphoreType.DMA])
    return pl.kernel(
        _edge_sc_body,
        out_type=jax.ShapeDtypeStruct((NC, NSLOT, A_PAD, LANES), jnp.float32),
        mesh=plsc.VectorSubcoreMesh(core_axis_name="c", subcore_axis_name="s",
                                    num_cores=NC, num_subcores=NS),
        compiler_params=pltpu.CompilerParams(use_tc_tiling_on_sc=False),
        scratch_types=scr,
    )


def kernel(R, Z, neighbor_idxs, box, W_emb):
    n_edges = neighbor_idxs.shape[1]
    R = R.astype(jnp.float32)
    tx = jnp.zeros((A_PAD,), jnp.float32).at[:N_ATOMS].set(R[:, 0])
    ty = jnp.zeros((A_PAD,), jnp.float32).at[:N_ATOMS].set(R[:, 1])
    tz = jnp.zeros((A_PAD,), jnp.float32).at[:N_ATOMS].set(R[:, 2])
    tzf = jnp.zeros((A_PAD,), jnp.float32).at[:N_ATOMS].set(Z.astype(jnp.float32))
    ie = jnp.zeros((E_PAD,), jnp.int32).at[:n_edges].set(neighbor_idxs[0])
    je = jnp.zeros((E_PAD,), jnp.int32).at[:n_edges].set(neighbor_idxs[1])
    wflat = W_emb.reshape(N_SPECIES * N_SPECIES, N_RADIAL, N_BASIS).astype(jnp.float32)
    ws = [jnp.zeros((N_SPECIES * N_SPECIES, LANES), jnp.float32)
          .at[:, :N_BASIS].set(wflat[:, r, :]) for r in range(N_RADIAL)]
    macc = _edge_sc()(tx, ty, tz, tzf, ie, je, *ws)
    out_t = _contract(macc)
    return jnp.transpose(out_t)[:N_ATOMS]
